# Initial kernel scaffold; baseline (speedup 1.0000x reference)
import jax
import jax.numpy as jnp
from jax.experimental import pallas as pl


def kernel(node_seq, edge_seq, edge_index, W_ih_n, W_hh_n, b_ih_n, b_hh_n, W_ih_e, W_hh_e, b_ih_e, b_hh_e, W_gcn1, b_gcn1, W_gcn2, b_gcn2, W_fc, b_fc):
    raise NotImplementedError("write your pallas kernel here")


# speedup vs baseline: 3.4590x; 3.4590x over previous
import functools

import jax
import jax.numpy as jnp
from jax import lax
from jax.experimental import pallas as pl
from jax.experimental.pallas import tpu as pltpu
from jax.experimental.pallas import tpu_sc as plsc

N, T, D = 10000, 8, 128
E, DE = 320000, 16
H = 256

NC, NS = 2, 16
E2 = E // NC
CH = 128
NCHUNK = E2 // CH
NITER = (NCHUNK + NS - 1) // NS
SLA = 624
TAILB = NS * SLA
TAILN = N - TAILB

def _sc_mesh():
    return plsc.VectorSubcoreMesh(core_axis_name="c", subcore_axis_name="s")


def _deg_partials(dst):
    @functools.partial(
        pl.kernel,
        out_type=jax.ShapeDtypeStruct((NC, N, 16), jnp.float32),
        mesh=_sc_mesh(),
        scratch_types=[
            pltpu.VMEM((CH,), jnp.int32),
            pltpu.VMEM((CH, 16), jnp.float32),
            pltpu.VMEM((CH, 16), jnp.float32),
            pltpu.VMEM_SHARED((N, 16), jnp.float32),
        ],
    )
    def k(dst_hbm, out_hbm, di_v, ones_v, zeros_v, acc_sh):
        cid = lax.axis_index("c")
        sid = lax.axis_index("s")

        @pl.loop(0, CH)
        def _(i):
            ones_v[i, :] = jnp.ones((16,), jnp.float32)

        zsrc = zeros_v
        @pl.loop(0, CH)
        def _(i):
            zsrc[i, :] = jnp.zeros((16,), jnp.float32)

        @pl.loop(0, SLA // CH)
        def _(i):
            pltpu.sync_copy(zsrc, acc_sh.at[pl.ds(sid * SLA + i * CH, CH)])
        rem = SLA - (SLA // CH) * CH
        if rem:
            pltpu.sync_copy(zsrc.at[pl.ds(0, rem)],
                            acc_sh.at[pl.ds(sid * SLA + (SLA // CH) * CH, rem)])

        @pl.when(sid == NS - 1)
        def _():
            pltpu.sync_copy(zsrc.at[pl.ds(0, TAILN)],
                            acc_sh.at[pl.ds(TAILB, TAILN)])

        plsc.subcore_barrier()

        @pl.loop(0, NITER)
        def _(i):
            ck = sid + i * NS

            @pl.when(ck < NCHUNK)
            def _():
                off = cid * E2 + ck * CH
                pltpu.sync_copy(dst_hbm.at[pl.ds(off, CH)], di_v)
                pltpu.sync_copy(ones_v, acc_sh.at[di_v], add=True)

        plsc.subcore_barrier()
        pltpu.sync_copy(acc_sh.at[pl.ds(sid * SLA, SLA)],
                        out_hbm.at[cid, pl.ds(sid * SLA, SLA)])

        @pl.when(sid == NS - 1)
        def _():
            pltpu.sync_copy(acc_sh.at[pl.ds(TAILB, TAILN)],
                            out_hbm.at[cid, pl.ds(TAILB, TAILN)])

    return k(dst)


def _seg_partials(table, src, dst):
    @functools.partial(
        pl.kernel,
        out_type=jax.ShapeDtypeStruct((NC, N, 128), jnp.float32),
        mesh=_sc_mesh(),
        scratch_types=[
            pltpu.VMEM((CH,), jnp.int32),
            pltpu.VMEM((CH,), jnp.int32),
            pltpu.VMEM((CH, 128), jnp.float32),
            pltpu.VMEM_SHARED((N, 128), jnp.float32),
            pltpu.SemaphoreType.DMA,
        ],
    )
    def k(tbl_hbm, src_hbm, dst_hbm, out_hbm, si_v, di_v, rows_v, acc_sh, sem):
        cid = lax.axis_index("c")
        sid = lax.axis_index("s")

        @pl.loop(0, CH)
        def _(i):
            @pl.loop(0, 128, step=16)
            def _(j):
                rows_v[i, pl.ds(j, 16)] = jnp.zeros((16,), jnp.float32)

        @pl.loop(0, SLA // CH)
        def _(i):
            pltpu.sync_copy(rows_v, acc_sh.at[pl.ds(sid * SLA + i * CH, CH)])
        rem = SLA - (SLA // CH) * CH
        if rem:
            pltpu.sync_copy(rows_v.at[pl.ds(0, rem)],
                            acc_sh.at[pl.ds(sid * SLA + (SLA // CH) * CH, rem)])

        @pl.when(sid == NS - 1)
        def _():
            pltpu.sync_copy(rows_v.at[pl.ds(0, TAILN)],
                            acc_sh.at[pl.ds(TAILB, TAILN)])

        plsc.subcore_barrier()

        @pl.loop(0, NITER)
        def _(i):
            ck = sid + i * NS

            @pl.when(ck < NCHUNK)
            def _():
                off = cid * E2 + ck * CH
                pltpu.sync_copy(src_hbm.at[pl.ds(off, CH)], si_v)
                pltpu.sync_copy(dst_hbm.at[pl.ds(off, CH)], di_v)
                pltpu.async_copy(tbl_hbm.at[si_v], rows_v, sem).wait()
                pltpu.sync_copy(rows_v, acc_sh.at[di_v], add=True)

        plsc.subcore_barrier()
        pltpu.sync_copy(acc_sh.at[pl.ds(sid * SLA, SLA)],
                        out_hbm.at[cid, pl.ds(sid * SLA, SLA)])

        @pl.when(sid == NS - 1)
        def _():
            pltpu.sync_copy(acc_sh.at[pl.ds(TAILB, TAILN)],
                            out_hbm.at[cid, pl.ds(TAILB, TAILN)])

    return k(table, src, dst)


def _node_stage(node_seq, degp, W_ihT, W_hhT, b_ih2, b_hh2, W1):
    BN = 1000

    def body(x_ref, dp_ref, wih_ref, whh_ref, bih_ref, bhh_ref, w1_ref,
             s1l_ref, s1r_ref, dv_ref):
        wih = wih_ref[...]
        whh = whh_ref[...]
        bih = bih_ref[...]
        bhh = bhh_ref[...]
        h = jnp.zeros((BN, D), jnp.float32)
        for t in range(T):
            xt = x_ref[:, t, :]
            gi = jnp.dot(xt, wih, preferred_element_type=jnp.float32) + bih
            gh = jnp.dot(h, whh, preferred_element_type=jnp.float32) + bhh
            r = jax.nn.sigmoid(gi[:, :D] + gh[:, :D])
            z = jax.nn.sigmoid(gi[:, D:2 * D] + gh[:, D:2 * D])
            n = jnp.tanh(gi[:, 2 * D:] + r * gh[:, 2 * D:])
            h = (1.0 - z) * n + z * h
        deg = dp_ref[0][:, 0:1] + dp_ref[1][:, 0:1] + 1.0
        dinv = lax.rsqrt(deg)
        s1 = jnp.dot(h, w1_ref[...], preferred_element_type=jnp.float32) * dinv
        s1l_ref[...] = s1[:, :128]
        s1r_ref[...] = s1[:, 128:]
        dv_ref[...] = jnp.broadcast_to(dinv, (BN, 128))

    return pl.pallas_call(
        body,
        grid=(N // BN,),
        in_specs=[
            pl.BlockSpec((BN, T, D), lambda i: (i, 0, 0)),
            pl.BlockSpec((NC, BN, 16), lambda i: (0, i, 0)),
            pl.BlockSpec((D, 3 * D), lambda i: (0, 0)),
            pl.BlockSpec((D, 3 * D), lambda i: (0, 0)),
            pl.BlockSpec((1, 3 * D), lambda i: (0, 0)),
            pl.BlockSpec((1, 3 * D), lambda i: (0, 0)),
            pl.BlockSpec((D, H), lambda i: (0, 0)),
        ],
        out_specs=[
            pl.BlockSpec((BN, 128), lambda i: (i, 0)),
            pl.BlockSpec((BN, 128), lambda i: (i, 0)),
            pl.BlockSpec((BN, 128), lambda i: (i, 0)),
        ],
        out_shape=[
            jax.ShapeDtypeStruct((N, 128), jnp.float32),
            jax.ShapeDtypeStruct((N, 128), jnp.float32),
            jax.ShapeDtypeStruct((N, 128), jnp.float32),
        ],
    )(node_seq, degp, W_ihT, W_hhT, b_ih2, b_hh2, W1)


def _mid_stage(g1pL, g1pR, s1L, s1R, dinvb, b1_2, W2):
    BN = 1000

    def body(gl_ref, gr_ref, sl_ref, sr_ref, dv_ref, b1_ref, w2_ref, o_ref):
        dinv = dv_ref[:, 0:1]
        b1 = b1_ref[...]
        hl = jnp.maximum(
            dinv * (gl_ref[0] + gl_ref[1] + sl_ref[...]) + b1[:, :128], 0.0)
        hr = jnp.maximum(
            dinv * (gr_ref[0] + gr_ref[1] + sr_ref[...]) + b1[:, 128:], 0.0)
        hcat = jnp.concatenate([hl, hr], axis=1)
        o_ref[...] = jnp.dot(hcat, w2_ref[...],
                             preferred_element_type=jnp.float32) * dinv

    return pl.pallas_call(
        body,
        grid=(N // BN,),
        in_specs=[
            pl.BlockSpec((NC, BN, 128), lambda i: (0, i, 0)),
            pl.BlockSpec((NC, BN, 128), lambda i: (0, i, 0)),
            pl.BlockSpec((BN, 128), lambda i: (i, 0)),
            pl.BlockSpec((BN, 128), lambda i: (i, 0)),
            pl.BlockSpec((BN, 128), lambda i: (i, 0)),
            pl.BlockSpec((1, H), lambda i: (0, 0)),
            pl.BlockSpec((H, D), lambda i: (0, 0)),
        ],
        out_specs=pl.BlockSpec((BN, 128), lambda i: (i, 0)),
        out_shape=jax.ShapeDtypeStruct((N, 128), jnp.float32),
    )(g1pL, g1pR, s1L, s1R, dinvb, b1_2, W2)


def _final_stage(g2p, s2, dinvb, b2_2):
    BN = 1000

    def body(g_ref, s_ref, dv_ref, b2_ref, o_ref):
        dinv = dv_ref[:, 0:1]
        o_ref[...] = dinv * (g_ref[0] + g_ref[1] + s_ref[...]) + b2_ref[...]

    return pl.pallas_call(
        body,
        grid=(N // BN,),
        in_specs=[
            pl.BlockSpec((NC, BN, 128), lambda i: (0, i, 0)),
            pl.BlockSpec((BN, 128), lambda i: (i, 0)),
            pl.BlockSpec((BN, 128), lambda i: (i, 0)),
            pl.BlockSpec((1, D), lambda i: (0, 0)),
        ],
        out_specs=pl.BlockSpec((BN, 128), lambda i: (i, 0)),
        out_shape=jax.ShapeDtypeStruct((N, D), jnp.float32),
    )(g2p, s2, dinvb, b2_2)


def _edge_stage(eseq3, W_ih16, W_hh16, b_i16, b_h16, W_fc16, b_fc16):
    BP = 400
    G = DE * DE

    def body(x_ref, wi_ref, wh_ref, bi_ref, bh_ref, wf_ref, bf_ref, o_ref):
        wi = wi_ref[...]
        wh = wh_ref[...]
        bi = bi_ref[...]
        bh = bh_ref[...]
        h = jnp.zeros((BP, G), jnp.float32)
        for t in range(T):
            xt = jnp.concatenate(
                [x_ref[:, j, DE * t:DE * (t + 1)] for j in range(16)], axis=1)
            gi = jnp.dot(xt, wi, preferred_element_type=jnp.float32) + bi
            gh = jnp.dot(h, wh, preferred_element_type=jnp.float32) + bh
            r = jax.nn.sigmoid(gi[:, :G] + gh[:, :G])
            z = jax.nn.sigmoid(gi[:, G:2 * G] + gh[:, G:2 * G])
            n = jnp.tanh(gi[:, 2 * G:] + r * gh[:, 2 * G:])
            h = (1.0 - z) * n + z * h
        o_ref[...] = jnp.dot(h, wf_ref[...],
                             preferred_element_type=jnp.float32) + bf_ref[...]

    return pl.pallas_call(
        body,
        grid=(E // 16 // BP,),
        in_specs=[
            pl.BlockSpec((BP, 16, 128), lambda i: (i, 0, 0)),
            pl.BlockSpec((G, 3 * G), lambda i: (0, 0)),
            pl.BlockSpec((G, 3 * G), lambda i: (0, 0)),
            pl.BlockSpec((1, 3 * G), lambda i: (0, 0)),
            pl.BlockSpec((1, 3 * G), lambda i: (0, 0)),
            pl.BlockSpec((G, G), lambda i: (0, 0)),
            pl.BlockSpec((1, G), lambda i: (0, 0)),
        ],
        out_specs=pl.BlockSpec((BP, G), lambda i: (i, 0)),
        out_shape=jax.ShapeDtypeStruct((E // 16, G), jnp.float32),
    )(eseq3, W_ih16, W_hh16, b_i16, b_h16, W_fc16, b_fc16)


def kernel(node_seq, edge_seq, edge_index,
           W_ih_n, W_hh_n, b_ih_n, b_hh_n,
           W_ih_e, W_hh_e, b_ih_e, b_hh_e,
           W_gcn1, b_gcn1, W_gcn2, b_gcn2, W_fc, b_fc):
    src = edge_index[0].astype(jnp.int32)
    dst = edge_index[1].astype(jnp.int32)

    W_ihT_n = W_ih_n.T
    W_hhT_n = W_hh_n.T
    b_ihn2 = b_ih_n.reshape(1, -1)
    b_hhn2 = b_hh_n.reshape(1, -1)

    eye16 = jnp.eye(16, dtype=jnp.float32)

    def blockdiag(w):
        return jnp.kron(eye16, w)

    W_ih16 = jnp.concatenate(
        [blockdiag(W_ih_e[16 * g:16 * (g + 1), :].T) for g in range(3)], axis=1)
    W_hh16 = jnp.concatenate(
        [blockdiag(W_hh_e[16 * g:16 * (g + 1), :].T) for g in range(3)], axis=1)
    b_i16 = jnp.concatenate(
        [jnp.tile(b_ih_e[16 * g:16 * (g + 1)], 16) for g in range(3)]
    ).reshape(1, -1)
    b_h16 = jnp.concatenate(
        [jnp.tile(b_hh_e[16 * g:16 * (g + 1)], 16) for g in range(3)]
    ).reshape(1, -1)
    W_fc16 = blockdiag(W_fc.T)
    b_fc16 = jnp.tile(b_fc, 16).reshape(1, -1)

    degp = _deg_partials(dst)
    s1L, s1R, dinvb = _node_stage(node_seq, degp, W_ihT_n, W_hhT_n,
                                  b_ihn2, b_hhn2, W_gcn1)
    g1pL = _seg_partials(s1L, src, dst)
    g1pR = _seg_partials(s1R, src, dst)
    s2 = _mid_stage(g1pL, g1pR, s1L, s1R, dinvb, b_gcn1.reshape(1, -1), W_gcn2)
    g2p = _seg_partials(s2, src, dst)
    x_rec = _final_stage(g2p, s2, dinvb, b_gcn2.reshape(1, -1))

    eseq3 = edge_seq.reshape(E // 16, 16, T * DE)
    e_pack = _edge_stage(eseq3, W_ih16, W_hh16, b_i16, b_h16, W_fc16, b_fc16)
    e_rec = e_pack.reshape(E, DE)

    return (x_rec, e_rec)


# speedup vs baseline: 8.5202x; 2.4632x over previous
import functools

import jax
import jax.numpy as jnp
from jax import lax
from jax.experimental import pallas as pl
from jax.experimental.pallas import tpu as pltpu
from jax.experimental.pallas import tpu_sc as plsc

N, T, D = 10000, 8, 128
E, DE = 320000, 16
H = 256

NC, NS = 2, 16
E2 = E // NC
CH = 128
NCHUNK = E2 // CH
NITER = (NCHUNK + NS - 1) // NS
SLA = 624
TAILB = NS * SLA
TAILN = N - TAILB

def _sc_mesh():
    return plsc.VectorSubcoreMesh(core_axis_name="c", subcore_axis_name="s")


def _deg_partials(dst):
    @functools.partial(
        pl.kernel,
        out_type=jax.ShapeDtypeStruct((NC, N, 16), jnp.float32),
        mesh=_sc_mesh(),
        scratch_types=[
            pltpu.VMEM((CH,), jnp.int32),
            pltpu.VMEM((CH, 16), jnp.float32),
            pltpu.VMEM((CH, 16), jnp.float32),
            pltpu.VMEM_SHARED((N, 16), jnp.float32),
        ],
    )
    def k(dst_hbm, out_hbm, di_v, ones_v, zeros_v, acc_sh):
        cid = lax.axis_index("c")
        sid = lax.axis_index("s")

        @pl.loop(0, CH)
        def _(i):
            ones_v[i, :] = jnp.ones((16,), jnp.float32)

        zsrc = zeros_v
        @pl.loop(0, CH)
        def _(i):
            zsrc[i, :] = jnp.zeros((16,), jnp.float32)

        @pl.loop(0, SLA // CH)
        def _(i):
            pltpu.sync_copy(zsrc, acc_sh.at[pl.ds(sid * SLA + i * CH, CH)])
        rem = SLA - (SLA // CH) * CH
        if rem:
            pltpu.sync_copy(zsrc.at[pl.ds(0, rem)],
                            acc_sh.at[pl.ds(sid * SLA + (SLA // CH) * CH, rem)])

        @pl.when(sid == NS - 1)
        def _():
            pltpu.sync_copy(zsrc.at[pl.ds(0, TAILN)],
                            acc_sh.at[pl.ds(TAILB, TAILN)])

        plsc.subcore_barrier()

        @pl.loop(0, NITER)
        def _(i):
            ck = sid + i * NS

            @pl.when(ck < NCHUNK)
            def _():
                off = cid * E2 + ck * CH
                pltpu.sync_copy(dst_hbm.at[pl.ds(off, CH)], di_v)
                pltpu.sync_copy(ones_v, acc_sh.at[di_v], add=True)

        plsc.subcore_barrier()
        pltpu.sync_copy(acc_sh.at[pl.ds(sid * SLA, SLA)],
                        out_hbm.at[cid, pl.ds(sid * SLA, SLA)])

        @pl.when(sid == NS - 1)
        def _():
            pltpu.sync_copy(acc_sh.at[pl.ds(TAILB, TAILN)],
                            out_hbm.at[cid, pl.ds(TAILB, TAILN)])

    return k(dst)


def _seg_partials(table, src, dst):
    @functools.partial(
        pl.kernel,
        out_type=jax.ShapeDtypeStruct((NC, N, 128), jnp.float32),
        mesh=_sc_mesh(),
        scratch_types=[
            pltpu.VMEM((CH,), jnp.int32),
            pltpu.VMEM((CH,), jnp.int32),
            pltpu.VMEM((CH, 128), jnp.float32),
            pltpu.VMEM_SHARED((N, 128), jnp.float32),
            pltpu.SemaphoreType.DMA,
        ],
    )
    def k(tbl_hbm, src_hbm, dst_hbm, out_hbm, si_v, di_v, rows_v, acc_sh, sem):
        cid = lax.axis_index("c")
        sid = lax.axis_index("s")

        @pl.loop(0, CH)
        def _(i):
            @pl.loop(0, 128, step=16)
            def _(j):
                rows_v[i, pl.ds(j, 16)] = jnp.zeros((16,), jnp.float32)

        @pl.loop(0, SLA // CH)
        def _(i):
            pltpu.sync_copy(rows_v, acc_sh.at[pl.ds(sid * SLA + i * CH, CH)])
        rem = SLA - (SLA // CH) * CH
        if rem:
            pltpu.sync_copy(rows_v.at[pl.ds(0, rem)],
                            acc_sh.at[pl.ds(sid * SLA + (SLA // CH) * CH, rem)])

        @pl.when(sid == NS - 1)
        def _():
            pltpu.sync_copy(rows_v.at[pl.ds(0, TAILN)],
                            acc_sh.at[pl.ds(TAILB, TAILN)])

        plsc.subcore_barrier()

        @pl.loop(0, NITER)
        def _(i):
            ck = sid + i * NS

            @pl.when(ck < NCHUNK)
            def _():
                off = cid * E2 + ck * CH
                pltpu.sync_copy(src_hbm.at[pl.ds(off, CH)], si_v)
                pltpu.sync_copy(dst_hbm.at[pl.ds(off, CH)], di_v)
                pltpu.async_copy(tbl_hbm.at[si_v], rows_v, sem).wait()
                pltpu.sync_copy(rows_v, acc_sh.at[di_v], add=True)

        plsc.subcore_barrier()
        pltpu.sync_copy(acc_sh.at[pl.ds(sid * SLA, SLA)],
                        out_hbm.at[cid, pl.ds(sid * SLA, SLA)])

        @pl.when(sid == NS - 1)
        def _():
            pltpu.sync_copy(acc_sh.at[pl.ds(TAILB, TAILN)],
                            out_hbm.at[cid, pl.ds(TAILB, TAILN)])

    return k(table, src, dst)


def _node_stage(node_seq, degp, W_ihT, W_hhT, b_ih2, b_hh2, W1):
    BN = 1000

    def body(x_ref, dp_ref, wih_ref, whh_ref, bih_ref, bhh_ref, w1_ref,
             s1l_ref, s1r_ref, dv_ref):
        wih = wih_ref[...]
        whh = whh_ref[...]
        bih = bih_ref[...]
        bhh = bhh_ref[...]
        h = jnp.zeros((BN, D), jnp.float32)
        for t in range(T):
            xt = x_ref[:, t, :]
            gi = jnp.dot(xt, wih, preferred_element_type=jnp.float32) + bih
            gh = jnp.dot(h, whh, preferred_element_type=jnp.float32) + bhh
            r = jax.nn.sigmoid(gi[:, :D] + gh[:, :D])
            z = jax.nn.sigmoid(gi[:, D:2 * D] + gh[:, D:2 * D])
            n = jnp.tanh(gi[:, 2 * D:] + r * gh[:, 2 * D:])
            h = (1.0 - z) * n + z * h
        deg = dp_ref[0][:, 0:1] + dp_ref[1][:, 0:1] + 1.0
        dinv = lax.rsqrt(deg)
        s1 = jnp.dot(h, w1_ref[...], preferred_element_type=jnp.float32) * dinv
        s1l_ref[...] = s1[:, :128]
        s1r_ref[...] = s1[:, 128:]
        dv_ref[...] = jnp.broadcast_to(dinv, (BN, 128))

    return pl.pallas_call(
        body,
        grid=(N // BN,),
        in_specs=[
            pl.BlockSpec((BN, T, D), lambda i: (i, 0, 0)),
            pl.BlockSpec((NC, BN, 16), lambda i: (0, i, 0)),
            pl.BlockSpec((D, 3 * D), lambda i: (0, 0)),
            pl.BlockSpec((D, 3 * D), lambda i: (0, 0)),
            pl.BlockSpec((1, 3 * D), lambda i: (0, 0)),
            pl.BlockSpec((1, 3 * D), lambda i: (0, 0)),
            pl.BlockSpec((D, H), lambda i: (0, 0)),
        ],
        out_specs=[
            pl.BlockSpec((BN, 128), lambda i: (i, 0)),
            pl.BlockSpec((BN, 128), lambda i: (i, 0)),
            pl.BlockSpec((BN, 128), lambda i: (i, 0)),
        ],
        out_shape=[
            jax.ShapeDtypeStruct((N, 128), jnp.float32),
            jax.ShapeDtypeStruct((N, 128), jnp.float32),
            jax.ShapeDtypeStruct((N, 128), jnp.float32),
        ],
    )(node_seq, degp, W_ihT, W_hhT, b_ih2, b_hh2, W1)


def _mid_stage(g1pL, g1pR, s1L, s1R, dinvb, b1_2, W2):
    BN = 1000

    def body(gl_ref, gr_ref, sl_ref, sr_ref, dv_ref, b1_ref, w2_ref, o_ref):
        dinv = dv_ref[:, 0:1]
        b1 = b1_ref[...]
        hl = jnp.maximum(
            dinv * (gl_ref[0] + gl_ref[1] + sl_ref[...]) + b1[:, :128], 0.0)
        hr = jnp.maximum(
            dinv * (gr_ref[0] + gr_ref[1] + sr_ref[...]) + b1[:, 128:], 0.0)
        hcat = jnp.concatenate([hl, hr], axis=1)
        o_ref[...] = jnp.dot(hcat, w2_ref[...],
                             preferred_element_type=jnp.float32) * dinv

    return pl.pallas_call(
        body,
        grid=(N // BN,),
        in_specs=[
            pl.BlockSpec((NC, BN, 128), lambda i: (0, i, 0)),
            pl.BlockSpec((NC, BN, 128), lambda i: (0, i, 0)),
            pl.BlockSpec((BN, 128), lambda i: (i, 0)),
            pl.BlockSpec((BN, 128), lambda i: (i, 0)),
            pl.BlockSpec((BN, 128), lambda i: (i, 0)),
            pl.BlockSpec((1, H), lambda i: (0, 0)),
            pl.BlockSpec((H, D), lambda i: (0, 0)),
        ],
        out_specs=pl.BlockSpec((BN, 128), lambda i: (i, 0)),
        out_shape=jax.ShapeDtypeStruct((N, 128), jnp.float32),
    )(g1pL, g1pR, s1L, s1R, dinvb, b1_2, W2)


def _final_stage(g2p, s2, dinvb, b2_2):
    BN = 1000

    def body(g_ref, s_ref, dv_ref, b2_ref, o_ref):
        dinv = dv_ref[:, 0:1]
        o_ref[...] = dinv * (g_ref[0] + g_ref[1] + s_ref[...]) + b2_ref[...]

    return pl.pallas_call(
        body,
        grid=(N // BN,),
        in_specs=[
            pl.BlockSpec((NC, BN, 128), lambda i: (0, i, 0)),
            pl.BlockSpec((BN, 128), lambda i: (i, 0)),
            pl.BlockSpec((BN, 128), lambda i: (i, 0)),
            pl.BlockSpec((1, D), lambda i: (0, 0)),
        ],
        out_specs=pl.BlockSpec((BN, 128), lambda i: (i, 0)),
        out_shape=jax.ShapeDtypeStruct((N, D), jnp.float32),
    )(g2p, s2, dinvb, b2_2)


def _edge_stage(eseq3, W_ih16, W_hh16, b_i16, b_h16, W_fc16, b_fc16):
    BE = 6400
    BP = BE // 16
    G = DE * DE

    def body(x_ref, wi_ref, wh_ref, bi_ref, bh_ref, wf_ref, bf_ref, o_ref):
        wi = wi_ref[...]
        wh = wh_ref[...]
        bi = bi_ref[...]
        bh = bh_ref[...]
        h = jnp.zeros((BP, G), jnp.float32)
        for t in range(T):
            xt = jnp.concatenate(
                [x_ref[pl.ds(j * BP, BP), DE * t:DE * (t + 1)]
                 for j in range(16)], axis=1)
            gi = jnp.dot(xt, wi, preferred_element_type=jnp.float32) + bi
            gh = jnp.dot(h, wh, preferred_element_type=jnp.float32) + bh
            r = jax.nn.sigmoid(gi[:, :G] + gh[:, :G])
            z = jax.nn.sigmoid(gi[:, G:2 * G] + gh[:, G:2 * G])
            n = jnp.tanh(gi[:, 2 * G:] + r * gh[:, 2 * G:])
            h = (1.0 - z) * n + z * h
        e = jnp.dot(h, wf_ref[...],
                    preferred_element_type=jnp.float32) + bf_ref[...]
        for j in range(16):
            o_ref[pl.ds(j * BP, BP), :] = e[:, DE * j:DE * (j + 1)]

    return pl.pallas_call(
        body,
        grid=(E // BE,),
        in_specs=[
            pl.BlockSpec((BE, 128), lambda i: (i, 0)),
            pl.BlockSpec((G, 3 * G), lambda i: (0, 0)),
            pl.BlockSpec((G, 3 * G), lambda i: (0, 0)),
            pl.BlockSpec((1, 3 * G), lambda i: (0, 0)),
            pl.BlockSpec((1, 3 * G), lambda i: (0, 0)),
            pl.BlockSpec((G, G), lambda i: (0, 0)),
            pl.BlockSpec((1, G), lambda i: (0, 0)),
        ],
        out_specs=pl.BlockSpec((BE, DE), lambda i: (i, 0)),
        out_shape=jax.ShapeDtypeStruct((E, DE), jnp.float32),
    )(eseq3, W_ih16, W_hh16, b_i16, b_h16, W_fc16, b_fc16)


def kernel(node_seq, edge_seq, edge_index,
           W_ih_n, W_hh_n, b_ih_n, b_hh_n,
           W_ih_e, W_hh_e, b_ih_e, b_hh_e,
           W_gcn1, b_gcn1, W_gcn2, b_gcn2, W_fc, b_fc):
    src = edge_index[0].astype(jnp.int32)
    dst = edge_index[1].astype(jnp.int32)

    W_ihT_n = W_ih_n.T
    W_hhT_n = W_hh_n.T
    b_ihn2 = b_ih_n.reshape(1, -1)
    b_hhn2 = b_hh_n.reshape(1, -1)

    eye16 = jnp.eye(16, dtype=jnp.float32)

    def blockdiag(w):
        return jnp.kron(eye16, w)

    W_ih16 = jnp.concatenate(
        [blockdiag(W_ih_e[16 * g:16 * (g + 1), :].T) for g in range(3)], axis=1)
    W_hh16 = jnp.concatenate(
        [blockdiag(W_hh_e[16 * g:16 * (g + 1), :].T) for g in range(3)], axis=1)
    b_i16 = jnp.concatenate(
        [jnp.tile(b_ih_e[16 * g:16 * (g + 1)], 16) for g in range(3)]
    ).reshape(1, -1)
    b_h16 = jnp.concatenate(
        [jnp.tile(b_hh_e[16 * g:16 * (g + 1)], 16) for g in range(3)]
    ).reshape(1, -1)
    W_fc16 = blockdiag(W_fc.T)
    b_fc16 = jnp.tile(b_fc, 16).reshape(1, -1)

    degp = _deg_partials(dst)
    s1L, s1R, dinvb = _node_stage(node_seq, degp, W_ihT_n, W_hhT_n,
                                  b_ihn2, b_hhn2, W_gcn1)
    g1pL = _seg_partials(s1L, src, dst)
    g1pR = _seg_partials(s1R, src, dst)
    s2 = _mid_stage(g1pL, g1pR, s1L, s1R, dinvb, b_gcn1.reshape(1, -1), W_gcn2)
    g2p = _seg_partials(s2, src, dst)
    x_rec = _final_stage(g2p, s2, dinvb, b_gcn2.reshape(1, -1))

    eseq2 = edge_seq.reshape(E, T * DE)
    e_rec = _edge_stage(eseq2, W_ih16, W_hh16, b_i16, b_h16, W_fc16, b_fc16)

    return (x_rec, e_rec)


# speedup vs baseline: 10.6026x; 1.2444x over previous
import functools

import jax
import jax.numpy as jnp
from jax import lax
from jax.experimental import pallas as pl
from jax.experimental.pallas import tpu as pltpu
from jax.experimental.pallas import tpu_sc as plsc

N, T, D = 10000, 8, 128
E, DE = 320000, 16
H = 256

NC, NS = 2, 16
E2 = E // NC
CH = 128
NCHUNK = E2 // CH
NITER = (NCHUNK + NS - 1) // NS
SLA = 624
TAILB = NS * SLA
TAILN = N - TAILB

def _sc_mesh():
    return plsc.VectorSubcoreMesh(core_axis_name="c", subcore_axis_name="s")


def _deg_partials(dst):
    @functools.partial(
        pl.kernel,
        out_type=jax.ShapeDtypeStruct((NC, N, 16), jnp.float32),
        mesh=_sc_mesh(),
        scratch_types=[
            pltpu.VMEM((CH,), jnp.int32),
            pltpu.VMEM((CH, 16), jnp.float32),
            pltpu.VMEM((CH, 16), jnp.float32),
            pltpu.VMEM_SHARED((N, 16), jnp.float32),
        ],
    )
    def k(dst_hbm, out_hbm, di_v, ones_v, zeros_v, acc_sh):
        cid = lax.axis_index("c")
        sid = lax.axis_index("s")

        @pl.loop(0, CH)
        def _(i):
            ones_v[i, :] = jnp.ones((16,), jnp.float32)

        zsrc = zeros_v
        @pl.loop(0, CH)
        def _(i):
            zsrc[i, :] = jnp.zeros((16,), jnp.float32)

        @pl.loop(0, SLA // CH)
        def _(i):
            pltpu.sync_copy(zsrc, acc_sh.at[pl.ds(sid * SLA + i * CH, CH)])
        rem = SLA - (SLA // CH) * CH
        if rem:
            pltpu.sync_copy(zsrc.at[pl.ds(0, rem)],
                            acc_sh.at[pl.ds(sid * SLA + (SLA // CH) * CH, rem)])

        @pl.when(sid == NS - 1)
        def _():
            pltpu.sync_copy(zsrc.at[pl.ds(0, TAILN)],
                            acc_sh.at[pl.ds(TAILB, TAILN)])

        plsc.subcore_barrier()

        @pl.loop(0, NITER)
        def _(i):
            ck = sid + i * NS

            @pl.when(ck < NCHUNK)
            def _():
                off = cid * E2 + ck * CH
                pltpu.sync_copy(dst_hbm.at[pl.ds(off, CH)], di_v)
                pltpu.sync_copy(ones_v, acc_sh.at[di_v], add=True)

        plsc.subcore_barrier()
        pltpu.sync_copy(acc_sh.at[pl.ds(sid * SLA, SLA)],
                        out_hbm.at[cid, pl.ds(sid * SLA, SLA)])

        @pl.when(sid == NS - 1)
        def _():
            pltpu.sync_copy(acc_sh.at[pl.ds(TAILB, TAILN)],
                            out_hbm.at[cid, pl.ds(TAILB, TAILN)])

    return k(dst)


def _seg_partials(table, src, dst):
    @functools.partial(
        pl.kernel,
        out_type=jax.ShapeDtypeStruct((NC, N, 128), jnp.float32),
        mesh=_sc_mesh(),
        scratch_types=[
            pltpu.VMEM((CH,), jnp.int32),
            pltpu.VMEM((CH,), jnp.int32),
            pltpu.VMEM((CH, 128), jnp.float32),
            pltpu.VMEM_SHARED((N, 128), jnp.float32),
            pltpu.SemaphoreType.DMA,
        ],
    )
    def k(tbl_hbm, src_hbm, dst_hbm, out_hbm, si_v, di_v, rows_v, acc_sh, sem):
        cid = lax.axis_index("c")
        sid = lax.axis_index("s")

        @pl.loop(0, CH)
        def _(i):
            @pl.loop(0, 128, step=16)
            def _(j):
                rows_v[i, pl.ds(j, 16)] = jnp.zeros((16,), jnp.float32)

        @pl.loop(0, SLA // CH)
        def _(i):
            pltpu.sync_copy(rows_v, acc_sh.at[pl.ds(sid * SLA + i * CH, CH)])
        rem = SLA - (SLA // CH) * CH
        if rem:
            pltpu.sync_copy(rows_v.at[pl.ds(0, rem)],
                            acc_sh.at[pl.ds(sid * SLA + (SLA // CH) * CH, rem)])

        @pl.when(sid == NS - 1)
        def _():
            pltpu.sync_copy(rows_v.at[pl.ds(0, TAILN)],
                            acc_sh.at[pl.ds(TAILB, TAILN)])

        plsc.subcore_barrier()

        @pl.loop(0, NITER)
        def _(i):
            ck = sid + i * NS

            @pl.when(ck < NCHUNK)
            def _():
                off = cid * E2 + ck * CH
                pltpu.sync_copy(src_hbm.at[pl.ds(off, CH)], si_v)
                pltpu.sync_copy(dst_hbm.at[pl.ds(off, CH)], di_v)
                pltpu.async_copy(tbl_hbm.at[si_v], rows_v, sem).wait()
                pltpu.sync_copy(rows_v, acc_sh.at[di_v], add=True)

        plsc.subcore_barrier()
        pltpu.sync_copy(acc_sh.at[pl.ds(sid * SLA, SLA)],
                        out_hbm.at[cid, pl.ds(sid * SLA, SLA)])

        @pl.when(sid == NS - 1)
        def _():
            pltpu.sync_copy(acc_sh.at[pl.ds(TAILB, TAILN)],
                            out_hbm.at[cid, pl.ds(TAILB, TAILN)])

    return k(table, src, dst)


def _node_stage(node_seq, degp, W_ihT, W_hhT, b_ih2, b_hh2, W1):
    BN = 1000

    def body(x_ref, dp_ref, wih_ref, whh_ref, bih_ref, bhh_ref, w1_ref,
             s1l_ref, s1r_ref, dv_ref):
        wih = wih_ref[...]
        whh = whh_ref[...]
        bih = bih_ref[...]
        bhh = bhh_ref[...]
        h = jnp.zeros((BN, D), jnp.float32)
        for t in range(T):
            xt = x_ref[:, t, :]
            gi = jnp.dot(xt, wih, preferred_element_type=jnp.float32) + bih
            gh = jnp.dot(h, whh, preferred_element_type=jnp.float32) + bhh
            r = jax.nn.sigmoid(gi[:, :D] + gh[:, :D])
            z = jax.nn.sigmoid(gi[:, D:2 * D] + gh[:, D:2 * D])
            n = jnp.tanh(gi[:, 2 * D:] + r * gh[:, 2 * D:])
            h = (1.0 - z) * n + z * h
        deg = dp_ref[0][:, 0:1] + dp_ref[1][:, 0:1] + 1.0
        dinv = lax.rsqrt(deg)
        s1 = jnp.dot(h, w1_ref[...], preferred_element_type=jnp.float32) * dinv
        s1l_ref[...] = s1[:, :128]
        s1r_ref[...] = s1[:, 128:]
        dv_ref[...] = jnp.broadcast_to(dinv, (BN, 128))

    return pl.pallas_call(
        body,
        grid=(N // BN,),
        in_specs=[
            pl.BlockSpec((BN, T, D), lambda i: (i, 0, 0)),
            pl.BlockSpec((NC, BN, 16), lambda i: (0, i, 0)),
            pl.BlockSpec((D, 3 * D), lambda i: (0, 0)),
            pl.BlockSpec((D, 3 * D), lambda i: (0, 0)),
            pl.BlockSpec((1, 3 * D), lambda i: (0, 0)),
            pl.BlockSpec((1, 3 * D), lambda i: (0, 0)),
            pl.BlockSpec((D, H), lambda i: (0, 0)),
        ],
        out_specs=[
            pl.BlockSpec((BN, 128), lambda i: (i, 0)),
            pl.BlockSpec((BN, 128), lambda i: (i, 0)),
            pl.BlockSpec((BN, 128), lambda i: (i, 0)),
        ],
        out_shape=[
            jax.ShapeDtypeStruct((N, 128), jnp.float32),
            jax.ShapeDtypeStruct((N, 128), jnp.float32),
            jax.ShapeDtypeStruct((N, 128), jnp.float32),
        ],
    )(node_seq, degp, W_ihT, W_hhT, b_ih2, b_hh2, W1)


def _mid_stage(g1pL, g1pR, s1L, s1R, dinvb, b1_2, W2):
    BN = 1000

    def body(gl_ref, gr_ref, sl_ref, sr_ref, dv_ref, b1_ref, w2_ref, o_ref):
        dinv = dv_ref[:, 0:1]
        b1 = b1_ref[...]
        hl = jnp.maximum(
            dinv * (gl_ref[0] + gl_ref[1] + sl_ref[...]) + b1[:, :128], 0.0)
        hr = jnp.maximum(
            dinv * (gr_ref[0] + gr_ref[1] + sr_ref[...]) + b1[:, 128:], 0.0)
        hcat = jnp.concatenate([hl, hr], axis=1)
        o_ref[...] = jnp.dot(hcat, w2_ref[...],
                             preferred_element_type=jnp.float32) * dinv

    return pl.pallas_call(
        body,
        grid=(N // BN,),
        in_specs=[
            pl.BlockSpec((NC, BN, 128), lambda i: (0, i, 0)),
            pl.BlockSpec((NC, BN, 128), lambda i: (0, i, 0)),
            pl.BlockSpec((BN, 128), lambda i: (i, 0)),
            pl.BlockSpec((BN, 128), lambda i: (i, 0)),
            pl.BlockSpec((BN, 128), lambda i: (i, 0)),
            pl.BlockSpec((1, H), lambda i: (0, 0)),
            pl.BlockSpec((H, D), lambda i: (0, 0)),
        ],
        out_specs=pl.BlockSpec((BN, 128), lambda i: (i, 0)),
        out_shape=jax.ShapeDtypeStruct((N, 128), jnp.float32),
    )(g1pL, g1pR, s1L, s1R, dinvb, b1_2, W2)


def _final_stage(g2p, s2, dinvb, b2_2):
    BN = 1000

    def body(g_ref, s_ref, dv_ref, b2_ref, o_ref):
        dinv = dv_ref[:, 0:1]
        o_ref[...] = dinv * (g_ref[0] + g_ref[1] + s_ref[...]) + b2_ref[...]

    return pl.pallas_call(
        body,
        grid=(N // BN,),
        in_specs=[
            pl.BlockSpec((NC, BN, 128), lambda i: (0, i, 0)),
            pl.BlockSpec((BN, 128), lambda i: (i, 0)),
            pl.BlockSpec((BN, 128), lambda i: (i, 0)),
            pl.BlockSpec((1, D), lambda i: (0, 0)),
        ],
        out_specs=pl.BlockSpec((BN, 128), lambda i: (i, 0)),
        out_shape=jax.ShapeDtypeStruct((N, D), jnp.float32),
    )(g2p, s2, dinvb, b2_2)


def _edge_stage(eseq2, W_ih16, W_hh16, b_i16, b_h16, W_fc16, b_fc16):
    BE = 6400
    BP = BE // 16
    G = DE * DE

    def body(x_ref, wi_ref, wh_ref, bi_ref, bh_ref, wf_ref, bf_ref, o_ref):
        wi = wi_ref[...]
        wh = wh_ref[...]
        bi = bi_ref[...]
        bh = bh_ref[...]
        h = jnp.zeros((BP, G), jnp.float32)
        for t in range(T):
            xt = jnp.concatenate(
                [x_ref[pl.ds(j * BP, BP), DE * t:DE * (t + 1)]
                 for j in range(16)], axis=1)
            gi = jnp.dot(xt, wi, preferred_element_type=jnp.float32) + bi
            gh = jnp.dot(h, wh, preferred_element_type=jnp.float32) + bh
            r = jax.nn.sigmoid(gi[:, :G] + gh[:, :G])
            z = jax.nn.sigmoid(gi[:, G:2 * G] + gh[:, G:2 * G])
            n = jnp.tanh(gi[:, 2 * G:] + r * gh[:, 2 * G:])
            h = (1.0 - z) * n + z * h
        e = jnp.dot(h, wf_ref[...],
                    preferred_element_type=jnp.float32) + bf_ref[...]
        o_ref[...] = e

    return pl.pallas_call(
        body,
        grid=(E // BE,),
        in_specs=[
            pl.BlockSpec((BE, 128), lambda i: (i, 0)),
            pl.BlockSpec((G, 3 * G), lambda i: (0, 0)),
            pl.BlockSpec((G, 3 * G), lambda i: (0, 0)),
            pl.BlockSpec((1, 3 * G), lambda i: (0, 0)),
            pl.BlockSpec((1, 3 * G), lambda i: (0, 0)),
            pl.BlockSpec((G, G), lambda i: (0, 0)),
            pl.BlockSpec((1, G), lambda i: (0, 0)),
        ],
        out_specs=pl.BlockSpec((BP, G), lambda i: (i, 0)),
        out_shape=jax.ShapeDtypeStruct((E // 16, G), jnp.float32),
    )(eseq2, W_ih16, W_hh16, b_i16, b_h16, W_fc16, b_fc16)


def kernel(node_seq, edge_seq, edge_index,
           W_ih_n, W_hh_n, b_ih_n, b_hh_n,
           W_ih_e, W_hh_e, b_ih_e, b_hh_e,
           W_gcn1, b_gcn1, W_gcn2, b_gcn2, W_fc, b_fc):
    src = edge_index[0].astype(jnp.int32)
    dst = edge_index[1].astype(jnp.int32)

    W_ihT_n = W_ih_n.T
    W_hhT_n = W_hh_n.T
    b_ihn2 = b_ih_n.reshape(1, -1)
    b_hhn2 = b_hh_n.reshape(1, -1)

    eye16 = jnp.eye(16, dtype=jnp.float32)

    def blockdiag(w):
        return jnp.kron(eye16, w)

    W_ih16 = jnp.concatenate(
        [blockdiag(W_ih_e[16 * g:16 * (g + 1), :].T) for g in range(3)], axis=1)
    W_hh16 = jnp.concatenate(
        [blockdiag(W_hh_e[16 * g:16 * (g + 1), :].T) for g in range(3)], axis=1)
    b_i16 = jnp.concatenate(
        [jnp.tile(b_ih_e[16 * g:16 * (g + 1)], 16) for g in range(3)]
    ).reshape(1, -1)
    b_h16 = jnp.concatenate(
        [jnp.tile(b_hh_e[16 * g:16 * (g + 1)], 16) for g in range(3)]
    ).reshape(1, -1)
    W_fc16 = blockdiag(W_fc.T)
    b_fc16 = jnp.tile(b_fc, 16).reshape(1, -1)

    degp = _deg_partials(dst)
    s1L, s1R, dinvb = _node_stage(node_seq, degp, W_ihT_n, W_hhT_n,
                                  b_ihn2, b_hhn2, W_gcn1)
    g1pL = _seg_partials(s1L, src, dst)
    g1pR = _seg_partials(s1R, src, dst)
    s2 = _mid_stage(g1pL, g1pR, s1L, s1R, dinvb, b_gcn1.reshape(1, -1), W_gcn2)
    g2p = _seg_partials(s2, src, dst)
    x_rec = _final_stage(g2p, s2, dinvb, b_gcn2.reshape(1, -1))

    eseq2 = edge_seq.reshape(E, T * DE)
    e_pack = _edge_stage(eseq2, W_ih16, W_hh16, b_i16, b_h16, W_fc16, b_fc16)
    NB = E // 6400
    e_rec = (e_pack.reshape(NB, 400, 16, DE).transpose(0, 2, 1, 3)
             .reshape(E, DE))

    return (x_rec, e_rec)


# speedup vs baseline: 11.6590x; 1.0996x over previous
import functools

import jax
import jax.numpy as jnp
from jax import lax
from jax.experimental import pallas as pl
from jax.experimental.pallas import tpu as pltpu
from jax.experimental.pallas import tpu_sc as plsc

N, T, D = 10000, 8, 128
E, DE = 320000, 16
H = 256

NC, NS = 2, 16
E2 = E // NC
CH = 80
CPT = E2 // NS // CH
NPAIR = (CPT - 1) // 2
SLA = 624
TAILB = NS * SLA
TAILN = N - TAILB

def _sc_mesh():
    return plsc.VectorSubcoreMesh(core_axis_name="c", subcore_axis_name="s")


def _deg_partials(dst):
    @functools.partial(
        pl.kernel,
        out_type=jax.ShapeDtypeStruct((NC, N, 16), jnp.float32),
        mesh=_sc_mesh(),
        scratch_types=[
            pltpu.VMEM((CH,), jnp.int32),
            pltpu.VMEM((CH, 16), jnp.float32),
            pltpu.VMEM((CH, 16), jnp.float32),
            pltpu.VMEM_SHARED((N, 16), jnp.float32),
        ],
    )
    def k(dst_hbm, out_hbm, di_v, ones_v, zeros_v, acc_sh):
        cid = lax.axis_index("c")
        sid = lax.axis_index("s")

        @pl.loop(0, CH)
        def _(i):
            ones_v[i, :] = jnp.ones((16,), jnp.float32)

        zsrc = zeros_v
        @pl.loop(0, CH)
        def _(i):
            zsrc[i, :] = jnp.zeros((16,), jnp.float32)

        @pl.loop(0, SLA // CH)
        def _(i):
            pltpu.sync_copy(zsrc, acc_sh.at[pl.ds(sid * SLA + i * CH, CH)])
        rem = SLA - (SLA // CH) * CH
        if rem:
            pltpu.sync_copy(zsrc.at[pl.ds(0, rem)],
                            acc_sh.at[pl.ds(sid * SLA + (SLA // CH) * CH, rem)])

        @pl.when(sid == NS - 1)
        def _():
            pltpu.sync_copy(zsrc.at[pl.ds(0, TAILN)],
                            acc_sh.at[pl.ds(TAILB, TAILN)])

        plsc.subcore_barrier()

        base = (cid * NS + sid) * CPT * CH

        @pl.loop(0, CPT)
        def _(i):
            pltpu.sync_copy(dst_hbm.at[pl.ds(base + i * CH, CH)], di_v)
            pltpu.sync_copy(ones_v, acc_sh.at[di_v], add=True)

        plsc.subcore_barrier()
        pltpu.sync_copy(acc_sh.at[pl.ds(sid * SLA, SLA)],
                        out_hbm.at[cid, pl.ds(sid * SLA, SLA)])

        @pl.when(sid == NS - 1)
        def _():
            pltpu.sync_copy(acc_sh.at[pl.ds(TAILB, TAILN)],
                            out_hbm.at[cid, pl.ds(TAILB, TAILN)])

    return k(dst)


def _seg_partials(table, src, dst):
    @functools.partial(
        pl.kernel,
        out_type=jax.ShapeDtypeStruct((NC, N, 128), jnp.float32),
        mesh=_sc_mesh(),
        scratch_types=[
            pltpu.VMEM((CH,), jnp.int32),
            pltpu.VMEM((CH,), jnp.int32),
            pltpu.VMEM((CH,), jnp.int32),
            pltpu.VMEM((CH,), jnp.int32),
            pltpu.VMEM((CH, 128), jnp.float32),
            pltpu.VMEM((CH, 128), jnp.float32),
            pltpu.VMEM_SHARED((N, 128), jnp.float32),
            pltpu.SemaphoreType.DMA,
            pltpu.SemaphoreType.DMA,
        ],
    )
    def k(tbl_hbm, src_hbm, dst_hbm, out_hbm, sa_v, da_v, sb_v, db_v,
          ra_v, rb_v, acc_sh, sema, semb):
        cid = lax.axis_index("c")
        sid = lax.axis_index("s")

        @pl.loop(0, CH)
        def _(i):
            @pl.loop(0, 128, step=16)
            def _(j):
                ra_v[i, pl.ds(j, 16)] = jnp.zeros((16,), jnp.float32)

        @pl.loop(0, SLA // CH)
        def _(i):
            pltpu.sync_copy(ra_v, acc_sh.at[pl.ds(sid * SLA + i * CH, CH)])
        rem = SLA - (SLA // CH) * CH
        if rem:
            pltpu.sync_copy(ra_v.at[pl.ds(0, rem)],
                            acc_sh.at[pl.ds(sid * SLA + (SLA // CH) * CH, rem)])

        @pl.when(sid == NS - 1)
        def _():
            pltpu.sync_copy(ra_v.at[pl.ds(0, TAILN)],
                            acc_sh.at[pl.ds(TAILB, TAILN)])

        plsc.subcore_barrier()

        base = (cid * NS + sid) * CPT * CH

        def load_idx(c, sv, dv):
            pltpu.sync_copy(src_hbm.at[pl.ds(base + c * CH, CH)], sv)
            pltpu.sync_copy(dst_hbm.at[pl.ds(base + c * CH, CH)], dv)

        def start_gather(sv, rv, sem):
            return pltpu.async_copy(tbl_hbm.at[sv], rv, sem)

        def finish(sv, dv, rv, sem):
            pltpu.make_async_copy(tbl_hbm.at[sv], rv, sem).wait()
            pltpu.sync_copy(rv, acc_sh.at[dv], add=True)

        load_idx(0, sa_v, da_v)
        start_gather(sa_v, ra_v, sema)

        @pl.loop(0, NPAIR)
        def _(i):
            c = 2 * i
            load_idx(c + 1, sb_v, db_v)
            start_gather(sb_v, rb_v, semb)
            finish(sa_v, da_v, ra_v, sema)
            load_idx(c + 2, sa_v, da_v)
            start_gather(sa_v, ra_v, sema)
            finish(sb_v, db_v, rb_v, semb)

        finish(sa_v, da_v, ra_v, sema)

        plsc.subcore_barrier()
        pltpu.sync_copy(acc_sh.at[pl.ds(sid * SLA, SLA)],
                        out_hbm.at[cid, pl.ds(sid * SLA, SLA)])

        @pl.when(sid == NS - 1)
        def _():
            pltpu.sync_copy(acc_sh.at[pl.ds(TAILB, TAILN)],
                            out_hbm.at[cid, pl.ds(TAILB, TAILN)])

    return k(table, src, dst)


def _node_stage(node_seq, degp, W_ihT, W_hhT, b_ih2, b_hh2, W1):
    BN = 1000

    def body(x_ref, dp_ref, wih_ref, whh_ref, bih_ref, bhh_ref, w1_ref,
             s1l_ref, s1r_ref, dv_ref):
        wih = wih_ref[...]
        whh = whh_ref[...]
        bih = bih_ref[...]
        bhh = bhh_ref[...]
        h = jnp.zeros((BN, D), jnp.float32)
        for t in range(T):
            xt = x_ref[:, t, :]
            gi = jnp.dot(xt, wih, preferred_element_type=jnp.float32) + bih
            gh = jnp.dot(h, whh, preferred_element_type=jnp.float32) + bhh
            r = jax.nn.sigmoid(gi[:, :D] + gh[:, :D])
            z = jax.nn.sigmoid(gi[:, D:2 * D] + gh[:, D:2 * D])
            n = jnp.tanh(gi[:, 2 * D:] + r * gh[:, 2 * D:])
            h = (1.0 - z) * n + z * h
        deg = dp_ref[0][:, 0:1] + dp_ref[1][:, 0:1] + 1.0
        dinv = lax.rsqrt(deg)
        s1 = jnp.dot(h, w1_ref[...], preferred_element_type=jnp.float32) * dinv
        s1l_ref[...] = s1[:, :128]
        s1r_ref[...] = s1[:, 128:]
        dv_ref[...] = jnp.broadcast_to(dinv, (BN, 128))

    return pl.pallas_call(
        body,
        grid=(N // BN,),
        in_specs=[
            pl.BlockSpec((BN, T, D), lambda i: (i, 0, 0)),
            pl.BlockSpec((NC, BN, 16), lambda i: (0, i, 0)),
            pl.BlockSpec((D, 3 * D), lambda i: (0, 0)),
            pl.BlockSpec((D, 3 * D), lambda i: (0, 0)),
            pl.BlockSpec((1, 3 * D), lambda i: (0, 0)),
            pl.BlockSpec((1, 3 * D), lambda i: (0, 0)),
            pl.BlockSpec((D, H), lambda i: (0, 0)),
        ],
        out_specs=[
            pl.BlockSpec((BN, 128), lambda i: (i, 0)),
            pl.BlockSpec((BN, 128), lambda i: (i, 0)),
            pl.BlockSpec((BN, 128), lambda i: (i, 0)),
        ],
        out_shape=[
            jax.ShapeDtypeStruct((N, 128), jnp.float32),
            jax.ShapeDtypeStruct((N, 128), jnp.float32),
            jax.ShapeDtypeStruct((N, 128), jnp.float32),
        ],
    )(node_seq, degp, W_ihT, W_hhT, b_ih2, b_hh2, W1)


def _mid_stage(g1pL, g1pR, s1L, s1R, dinvb, b1_2, W2):
    BN = 1000

    def body(gl_ref, gr_ref, sl_ref, sr_ref, dv_ref, b1_ref, w2_ref, o_ref):
        dinv = dv_ref[:, 0:1]
        b1 = b1_ref[...]
        hl = jnp.maximum(
            dinv * (gl_ref[0] + gl_ref[1] + sl_ref[...]) + b1[:, :128], 0.0)
        hr = jnp.maximum(
            dinv * (gr_ref[0] + gr_ref[1] + sr_ref[...]) + b1[:, 128:], 0.0)
        hcat = jnp.concatenate([hl, hr], axis=1)
        o_ref[...] = jnp.dot(hcat, w2_ref[...],
                             preferred_element_type=jnp.float32) * dinv

    return pl.pallas_call(
        body,
        grid=(N // BN,),
        in_specs=[
            pl.BlockSpec((NC, BN, 128), lambda i: (0, i, 0)),
            pl.BlockSpec((NC, BN, 128), lambda i: (0, i, 0)),
            pl.BlockSpec((BN, 128), lambda i: (i, 0)),
            pl.BlockSpec((BN, 128), lambda i: (i, 0)),
            pl.BlockSpec((BN, 128), lambda i: (i, 0)),
            pl.BlockSpec((1, H), lambda i: (0, 0)),
            pl.BlockSpec((H, D), lambda i: (0, 0)),
        ],
        out_specs=pl.BlockSpec((BN, 128), lambda i: (i, 0)),
        out_shape=jax.ShapeDtypeStruct((N, 128), jnp.float32),
    )(g1pL, g1pR, s1L, s1R, dinvb, b1_2, W2)


def _final_stage(g2p, s2, dinvb, b2_2):
    BN = 1000

    def body(g_ref, s_ref, dv_ref, b2_ref, o_ref):
        dinv = dv_ref[:, 0:1]
        o_ref[...] = dinv * (g_ref[0] + g_ref[1] + s_ref[...]) + b2_ref[...]

    return pl.pallas_call(
        body,
        grid=(N // BN,),
        in_specs=[
            pl.BlockSpec((NC, BN, 128), lambda i: (0, i, 0)),
            pl.BlockSpec((BN, 128), lambda i: (i, 0)),
            pl.BlockSpec((BN, 128), lambda i: (i, 0)),
            pl.BlockSpec((1, D), lambda i: (0, 0)),
        ],
        out_specs=pl.BlockSpec((BN, 128), lambda i: (i, 0)),
        out_shape=jax.ShapeDtypeStruct((N, D), jnp.float32),
    )(g2p, s2, dinvb, b2_2)


def _edge_stage(eseq2, W_ih16, W_hh16, b_i16, b_h16, W_fc16, b_fc16):
    BE = 6400
    BP = BE // 16
    G = DE * DE

    def body(x_ref, wi_ref, wh_ref, bi_ref, bh_ref, wf_ref, bf_ref, o_ref):
        wi = wi_ref[...]
        wh = wh_ref[...]
        bi = bi_ref[...]
        bh = bh_ref[...]
        h = jnp.zeros((BP, G), jnp.float32)
        for t in range(T):
            xt = jnp.concatenate(
                [x_ref[pl.ds(j * BP, BP), DE * t:DE * (t + 1)]
                 for j in range(16)], axis=1)
            gi = jnp.dot(xt, wi, preferred_element_type=jnp.float32) + bi
            gh = jnp.dot(h, wh, preferred_element_type=jnp.float32) + bh
            r = jax.nn.sigmoid(gi[:, :G] + gh[:, :G])
            z = jax.nn.sigmoid(gi[:, G:2 * G] + gh[:, G:2 * G])
            n = jnp.tanh(gi[:, 2 * G:] + r * gh[:, 2 * G:])
            h = (1.0 - z) * n + z * h
        e = jnp.dot(h, wf_ref[...],
                    preferred_element_type=jnp.float32) + bf_ref[...]
        o_ref[...] = e

    return pl.pallas_call(
        body,
        grid=(E // BE,),
        in_specs=[
            pl.BlockSpec((BE, 128), lambda i: (i, 0)),
            pl.BlockSpec((G, 3 * G), lambda i: (0, 0)),
            pl.BlockSpec((G, 3 * G), lambda i: (0, 0)),
            pl.BlockSpec((1, 3 * G), lambda i: (0, 0)),
            pl.BlockSpec((1, 3 * G), lambda i: (0, 0)),
            pl.BlockSpec((G, G), lambda i: (0, 0)),
            pl.BlockSpec((1, G), lambda i: (0, 0)),
        ],
        out_specs=pl.BlockSpec((BP, G), lambda i: (i, 0)),
        out_shape=jax.ShapeDtypeStruct((E // 16, G), jnp.float32),
    )(eseq2, W_ih16, W_hh16, b_i16, b_h16, W_fc16, b_fc16)


def kernel(node_seq, edge_seq, edge_index,
           W_ih_n, W_hh_n, b_ih_n, b_hh_n,
           W_ih_e, W_hh_e, b_ih_e, b_hh_e,
           W_gcn1, b_gcn1, W_gcn2, b_gcn2, W_fc, b_fc):
    src = edge_index[0].astype(jnp.int32)
    dst = edge_index[1].astype(jnp.int32)

    W_ihT_n = W_ih_n.T
    W_hhT_n = W_hh_n.T
    b_ihn2 = b_ih_n.reshape(1, -1)
    b_hhn2 = b_hh_n.reshape(1, -1)

    eye16 = jnp.eye(16, dtype=jnp.float32)

    def blockdiag(w):
        return jnp.kron(eye16, w)

    W_ih16 = jnp.concatenate(
        [blockdiag(W_ih_e[16 * g:16 * (g + 1), :].T) for g in range(3)], axis=1)
    W_hh16 = jnp.concatenate(
        [blockdiag(W_hh_e[16 * g:16 * (g + 1), :].T) for g in range(3)], axis=1)
    b_i16 = jnp.concatenate(
        [jnp.tile(b_ih_e[16 * g:16 * (g + 1)], 16) for g in range(3)]
    ).reshape(1, -1)
    b_h16 = jnp.concatenate(
        [jnp.tile(b_hh_e[16 * g:16 * (g + 1)], 16) for g in range(3)]
    ).reshape(1, -1)
    W_fc16 = blockdiag(W_fc.T)
    b_fc16 = jnp.tile(b_fc, 16).reshape(1, -1)

    degp = _deg_partials(dst)
    s1L, s1R, dinvb = _node_stage(node_seq, degp, W_ihT_n, W_hhT_n,
                                  b_ihn2, b_hhn2, W_gcn1)
    g1pL = _seg_partials(s1L, src, dst)
    g1pR = _seg_partials(s1R, src, dst)
    s2 = _mid_stage(g1pL, g1pR, s1L, s1R, dinvb, b_gcn1.reshape(1, -1), W_gcn2)
    g2p = _seg_partials(s2, src, dst)
    x_rec = _final_stage(g2p, s2, dinvb, b_gcn2.reshape(1, -1))

    eseq2 = edge_seq.reshape(E, T * DE)
    e_pack = _edge_stage(eseq2, W_ih16, W_hh16, b_i16, b_h16, W_fc16, b_fc16)
    NB = E // 6400
    e_rec = (e_pack.reshape(NB, 400, 16, DE).transpose(0, 2, 1, 3)
             .reshape(E, DE))

    return (x_rec, e_rec)


# speedup vs baseline: 12.2122x; 1.0474x over previous
import functools

import jax
import jax.numpy as jnp
from jax import lax
from jax.experimental import pallas as pl
from jax.experimental.pallas import tpu as pltpu
from jax.experimental.pallas import tpu_sc as plsc

N, T, D = 10000, 8, 128
E, DE = 320000, 16
H = 256

NC, NS = 2, 16
E2 = E // NC
CH = 80
CPT = E2 // NS // CH
NPAIR = (CPT - 1) // 2
SLA = 624
TAILB = NS * SLA
TAILN = N - TAILB

def _sc_mesh():
    return plsc.VectorSubcoreMesh(core_axis_name="c", subcore_axis_name="s")


def _deg_partials(dst):
    @functools.partial(
        pl.kernel,
        out_type=jax.ShapeDtypeStruct((NC, N, 16), jnp.float32),
        mesh=_sc_mesh(),
        scratch_types=[
            pltpu.VMEM((CH,), jnp.int32),
            pltpu.VMEM((CH, 16), jnp.float32),
            pltpu.VMEM((CH, 16), jnp.float32),
            pltpu.VMEM_SHARED((N, 16), jnp.float32),
        ],
    )
    def k(dst_hbm, out_hbm, di_v, ones_v, zeros_v, acc_sh):
        cid = lax.axis_index("c")
        sid = lax.axis_index("s")

        @pl.loop(0, CH)
        def _(i):
            ones_v[i, :] = jnp.ones((16,), jnp.float32)

        zsrc = zeros_v
        @pl.loop(0, CH)
        def _(i):
            zsrc[i, :] = jnp.zeros((16,), jnp.float32)

        @pl.loop(0, SLA // CH)
        def _(i):
            pltpu.sync_copy(zsrc, acc_sh.at[pl.ds(sid * SLA + i * CH, CH)])
        rem = SLA - (SLA // CH) * CH
        if rem:
            pltpu.sync_copy(zsrc.at[pl.ds(0, rem)],
                            acc_sh.at[pl.ds(sid * SLA + (SLA // CH) * CH, rem)])

        @pl.when(sid == NS - 1)
        def _():
            pltpu.sync_copy(zsrc.at[pl.ds(0, TAILN)],
                            acc_sh.at[pl.ds(TAILB, TAILN)])

        plsc.subcore_barrier()

        base = (cid * NS + sid) * CPT * CH

        @pl.loop(0, CPT)
        def _(i):
            pltpu.sync_copy(dst_hbm.at[pl.ds(base + i * CH, CH)], di_v)
            pltpu.sync_copy(ones_v, acc_sh.at[di_v], add=True)

        plsc.subcore_barrier()
        pltpu.sync_copy(acc_sh.at[pl.ds(sid * SLA, SLA)],
                        out_hbm.at[cid, pl.ds(sid * SLA, SLA)])

        @pl.when(sid == NS - 1)
        def _():
            pltpu.sync_copy(acc_sh.at[pl.ds(TAILB, TAILN)],
                            out_hbm.at[cid, pl.ds(TAILB, TAILN)])

    return k(dst)


def _seg_partials(table, src, dst):
    @functools.partial(
        pl.kernel,
        out_type=jax.ShapeDtypeStruct((NC, N, 128), jnp.float32),
        mesh=_sc_mesh(),
        scratch_types=[
            pltpu.VMEM((CH,), jnp.int32),
            pltpu.VMEM((CH,), jnp.int32),
            pltpu.VMEM((CH,), jnp.int32),
            pltpu.VMEM((CH,), jnp.int32),
            pltpu.VMEM((CH, 128), jnp.float32),
            pltpu.VMEM((CH, 128), jnp.float32),
            pltpu.VMEM_SHARED((N, 128), jnp.float32),
            pltpu.SemaphoreType.DMA,
            pltpu.SemaphoreType.DMA,
        ],
    )
    def k(tbl_hbm, src_hbm, dst_hbm, out_hbm, sa_v, da_v, sb_v, db_v,
          ra_v, rb_v, acc_sh, sema, semb):
        cid = lax.axis_index("c")
        sid = lax.axis_index("s")

        @pl.loop(0, CH)
        def _(i):
            @pl.loop(0, 128, step=16)
            def _(j):
                ra_v[i, pl.ds(j, 16)] = jnp.zeros((16,), jnp.float32)

        @pl.loop(0, SLA // CH)
        def _(i):
            pltpu.sync_copy(ra_v, acc_sh.at[pl.ds(sid * SLA + i * CH, CH)])
        rem = SLA - (SLA // CH) * CH
        if rem:
            pltpu.sync_copy(ra_v.at[pl.ds(0, rem)],
                            acc_sh.at[pl.ds(sid * SLA + (SLA // CH) * CH, rem)])

        @pl.when(sid == NS - 1)
        def _():
            pltpu.sync_copy(ra_v.at[pl.ds(0, TAILN)],
                            acc_sh.at[pl.ds(TAILB, TAILN)])

        plsc.subcore_barrier()

        base = (cid * NS + sid) * CPT * CH

        def load_idx(c, sv, dv):
            pltpu.sync_copy(src_hbm.at[pl.ds(base + c * CH, CH)], sv)
            pltpu.sync_copy(dst_hbm.at[pl.ds(base + c * CH, CH)], dv)

        def start_gather(sv, rv, sem):
            return pltpu.async_copy(tbl_hbm.at[sv], rv, sem)

        def finish(sv, dv, rv, sem):
            pltpu.make_async_copy(tbl_hbm.at[sv], rv, sem).wait()
            pltpu.sync_copy(rv, acc_sh.at[dv], add=True)

        load_idx(0, sa_v, da_v)
        start_gather(sa_v, ra_v, sema)

        @pl.loop(0, NPAIR)
        def _(i):
            c = 2 * i
            load_idx(c + 1, sb_v, db_v)
            start_gather(sb_v, rb_v, semb)
            finish(sa_v, da_v, ra_v, sema)
            load_idx(c + 2, sa_v, da_v)
            start_gather(sa_v, ra_v, sema)
            finish(sb_v, db_v, rb_v, semb)

        finish(sa_v, da_v, ra_v, sema)

        plsc.subcore_barrier()
        pltpu.sync_copy(acc_sh.at[pl.ds(sid * SLA, SLA)],
                        out_hbm.at[cid, pl.ds(sid * SLA, SLA)])

        @pl.when(sid == NS - 1)
        def _():
            pltpu.sync_copy(acc_sh.at[pl.ds(TAILB, TAILN)],
                            out_hbm.at[cid, pl.ds(TAILB, TAILN)])

    return k(table, src, dst)


def _node_stage(node_seq, W_ihT, W_hhT, b_ih2, b_hh2, W1):
    BN = 1000

    def body(x_ref, wih_ref, whh_ref, bih_ref, bhh_ref, w1_ref,
             xl_ref, xr_ref):
        wih = wih_ref[...]
        whh = whh_ref[...]
        bih = bih_ref[...]
        bhh = bhh_ref[...]
        h = jnp.zeros((BN, D), jnp.float32)
        for t in range(T):
            xt = x_ref[:, t, :]
            gi = jnp.dot(xt, wih, preferred_element_type=jnp.float32) + bih
            gh = jnp.dot(h, whh, preferred_element_type=jnp.float32) + bhh
            r = jax.nn.sigmoid(gi[:, :D] + gh[:, :D])
            z = jax.nn.sigmoid(gi[:, D:2 * D] + gh[:, D:2 * D])
            n = jnp.tanh(gi[:, 2 * D:] + r * gh[:, 2 * D:])
            h = (1.0 - z) * n + z * h
        xw = jnp.dot(h, w1_ref[...], preferred_element_type=jnp.float32)
        xl_ref[...] = xw[:, :128]
        xr_ref[...] = xw[:, 128:]

    return pl.pallas_call(
        body,
        grid=(N // BN,),
        in_specs=[
            pl.BlockSpec((BN, T, D), lambda i: (i, 0, 0)),
            pl.BlockSpec((D, 3 * D), lambda i: (0, 0)),
            pl.BlockSpec((D, 3 * D), lambda i: (0, 0)),
            pl.BlockSpec((1, 3 * D), lambda i: (0, 0)),
            pl.BlockSpec((1, 3 * D), lambda i: (0, 0)),
            pl.BlockSpec((D, H), lambda i: (0, 0)),
        ],
        out_specs=[
            pl.BlockSpec((BN, 128), lambda i: (i, 0)),
            pl.BlockSpec((BN, 128), lambda i: (i, 0)),
        ],
        out_shape=[
            jax.ShapeDtypeStruct((N, 128), jnp.float32),
            jax.ShapeDtypeStruct((N, 128), jnp.float32),
        ],
    )(node_seq, W_ihT, W_hhT, b_ih2, b_hh2, W1)


def _scale_stage(xw1L, xw1R, degp):
    BN = 1000

    def body(xl_ref, xr_ref, dp_ref, sl_ref, sr_ref, dv_ref):
        deg = dp_ref[0][:, 0:1] + dp_ref[1][:, 0:1] + 1.0
        dinv = lax.rsqrt(deg)
        sl_ref[...] = xl_ref[...] * dinv
        sr_ref[...] = xr_ref[...] * dinv
        dv_ref[...] = jnp.broadcast_to(dinv, (BN, 128))

    return pl.pallas_call(
        body,
        grid=(N // BN,),
        in_specs=[
            pl.BlockSpec((BN, 128), lambda i: (i, 0)),
            pl.BlockSpec((BN, 128), lambda i: (i, 0)),
            pl.BlockSpec((NC, BN, 16), lambda i: (0, i, 0)),
        ],
        out_specs=[
            pl.BlockSpec((BN, 128), lambda i: (i, 0)),
            pl.BlockSpec((BN, 128), lambda i: (i, 0)),
            pl.BlockSpec((BN, 128), lambda i: (i, 0)),
        ],
        out_shape=[
            jax.ShapeDtypeStruct((N, 128), jnp.float32),
            jax.ShapeDtypeStruct((N, 128), jnp.float32),
            jax.ShapeDtypeStruct((N, 128), jnp.float32),
        ],
    )(xw1L, xw1R, degp)


def _mid_stage(g1pL, g1pR, s1L, s1R, dinvb, b1_2, W2):
    BN = 1000

    def body(gl_ref, gr_ref, sl_ref, sr_ref, dv_ref, b1_ref, w2_ref, o_ref):
        dinv = dv_ref[:, 0:1]
        b1 = b1_ref[...]
        hl = jnp.maximum(
            dinv * (gl_ref[0] + gl_ref[1] + sl_ref[...]) + b1[:, :128], 0.0)
        hr = jnp.maximum(
            dinv * (gr_ref[0] + gr_ref[1] + sr_ref[...]) + b1[:, 128:], 0.0)
        hcat = jnp.concatenate([hl, hr], axis=1)
        o_ref[...] = jnp.dot(hcat, w2_ref[...],
                             preferred_element_type=jnp.float32) * dinv

    return pl.pallas_call(
        body,
        grid=(N // BN,),
        in_specs=[
            pl.BlockSpec((NC, BN, 128), lambda i: (0, i, 0)),
            pl.BlockSpec((NC, BN, 128), lambda i: (0, i, 0)),
            pl.BlockSpec((BN, 128), lambda i: (i, 0)),
            pl.BlockSpec((BN, 128), lambda i: (i, 0)),
            pl.BlockSpec((BN, 128), lambda i: (i, 0)),
            pl.BlockSpec((1, H), lambda i: (0, 0)),
            pl.BlockSpec((H, D), lambda i: (0, 0)),
        ],
        out_specs=pl.BlockSpec((BN, 128), lambda i: (i, 0)),
        out_shape=jax.ShapeDtypeStruct((N, 128), jnp.float32),
    )(g1pL, g1pR, s1L, s1R, dinvb, b1_2, W2)


def _final_stage(g2p, s2, dinvb, b2_2):
    BN = 1000

    def body(g_ref, s_ref, dv_ref, b2_ref, o_ref):
        dinv = dv_ref[:, 0:1]
        o_ref[...] = dinv * (g_ref[0] + g_ref[1] + s_ref[...]) + b2_ref[...]

    return pl.pallas_call(
        body,
        grid=(N // BN,),
        in_specs=[
            pl.BlockSpec((NC, BN, 128), lambda i: (0, i, 0)),
            pl.BlockSpec((BN, 128), lambda i: (i, 0)),
            pl.BlockSpec((BN, 128), lambda i: (i, 0)),
            pl.BlockSpec((1, D), lambda i: (0, 0)),
        ],
        out_specs=pl.BlockSpec((BN, 128), lambda i: (i, 0)),
        out_shape=jax.ShapeDtypeStruct((N, D), jnp.float32),
    )(g2p, s2, dinvb, b2_2)


def _edge_stage(eseq2, W_ih16, W_hh16, b_i16, b_h16, W_fc16, b_fc16):
    BE = 6400
    BP = BE // 16
    G = DE * DE

    def body(x_ref, wi_ref, wh_ref, bi_ref, bh_ref, wf_ref, bf_ref, o_ref):
        wi = wi_ref[...]
        wh = wh_ref[...]
        bi = bi_ref[...]
        bh = bh_ref[...]
        h = jnp.zeros((BP, G), jnp.float32)
        for t in range(T):
            xt = jnp.concatenate(
                [x_ref[pl.ds(j * BP, BP), DE * t:DE * (t + 1)]
                 for j in range(16)], axis=1)
            gi = jnp.dot(xt, wi, preferred_element_type=jnp.float32) + bi
            gh = jnp.dot(h, wh, preferred_element_type=jnp.float32) + bh
            r = jax.nn.sigmoid(gi[:, :G] + gh[:, :G])
            z = jax.nn.sigmoid(gi[:, G:2 * G] + gh[:, G:2 * G])
            n = jnp.tanh(gi[:, 2 * G:] + r * gh[:, 2 * G:])
            h = (1.0 - z) * n + z * h
        e = jnp.dot(h, wf_ref[...],
                    preferred_element_type=jnp.float32) + bf_ref[...]
        o_ref[...] = e

    return pl.pallas_call(
        body,
        grid=(E // BE,),
        in_specs=[
            pl.BlockSpec((BE, 128), lambda i: (i, 0)),
            pl.BlockSpec((G, 3 * G), lambda i: (0, 0)),
            pl.BlockSpec((G, 3 * G), lambda i: (0, 0)),
            pl.BlockSpec((1, 3 * G), lambda i: (0, 0)),
            pl.BlockSpec((1, 3 * G), lambda i: (0, 0)),
            pl.BlockSpec((G, G), lambda i: (0, 0)),
            pl.BlockSpec((1, G), lambda i: (0, 0)),
        ],
        out_specs=pl.BlockSpec((BP, G), lambda i: (i, 0)),
        out_shape=jax.ShapeDtypeStruct((E // 16, G), jnp.float32),
    )(eseq2, W_ih16, W_hh16, b_i16, b_h16, W_fc16, b_fc16)


def kernel(node_seq, edge_seq, edge_index,
           W_ih_n, W_hh_n, b_ih_n, b_hh_n,
           W_ih_e, W_hh_e, b_ih_e, b_hh_e,
           W_gcn1, b_gcn1, W_gcn2, b_gcn2, W_fc, b_fc):
    src = edge_index[0].astype(jnp.int32)
    dst = edge_index[1].astype(jnp.int32)

    W_ihT_n = W_ih_n.T
    W_hhT_n = W_hh_n.T
    b_ihn2 = b_ih_n.reshape(1, -1)
    b_hhn2 = b_hh_n.reshape(1, -1)

    eye16 = jnp.eye(16, dtype=jnp.float32)

    def blockdiag(w):
        return jnp.kron(eye16, w)

    W_ih16 = jnp.concatenate(
        [blockdiag(W_ih_e[16 * g:16 * (g + 1), :].T) for g in range(3)], axis=1)
    W_hh16 = jnp.concatenate(
        [blockdiag(W_hh_e[16 * g:16 * (g + 1), :].T) for g in range(3)], axis=1)
    b_i16 = jnp.concatenate(
        [jnp.tile(b_ih_e[16 * g:16 * (g + 1)], 16) for g in range(3)]
    ).reshape(1, -1)
    b_h16 = jnp.concatenate(
        [jnp.tile(b_hh_e[16 * g:16 * (g + 1)], 16) for g in range(3)]
    ).reshape(1, -1)
    W_fc16 = blockdiag(W_fc.T)
    b_fc16 = jnp.tile(b_fc, 16).reshape(1, -1)

    degp = _deg_partials(dst)
    xw1L, xw1R = _node_stage(node_seq, W_ihT_n, W_hhT_n,
                             b_ihn2, b_hhn2, W_gcn1)
    s1L, s1R, dinvb = _scale_stage(xw1L, xw1R, degp)
    g1pL = _seg_partials(s1L, src, dst)
    g1pR = _seg_partials(s1R, src, dst)
    s2 = _mid_stage(g1pL, g1pR, s1L, s1R, dinvb, b_gcn1.reshape(1, -1), W_gcn2)
    g2p = _seg_partials(s2, src, dst)
    x_rec = _final_stage(g2p, s2, dinvb, b_gcn2.reshape(1, -1))

    eseq2 = edge_seq.reshape(E, T * DE)
    e_pack = _edge_stage(eseq2, W_ih16, W_hh16, b_i16, b_h16, W_fc16, b_fc16)
    NB = E // 6400
    e_rec = (e_pack.reshape(NB, 400, 16, DE).transpose(0, 2, 1, 3)
             .reshape(E, DE))

    return (x_rec, e_rec)
